# baseline (speedup 1.0000x reference)
import jax
import jax.numpy as jnp
from jax import lax
from jax.experimental import pallas as pl

_BB = 32
_NFILL = 2


def _bcast_kernel(dummy_ref, pos_ref, out_ref):
    del dummy_ref
    i = pl.program_id(0)
    n = pl.num_programs(0)
    steps_per_plane = n // pos_ref.shape[0]

    @pl.when(lax.rem(i, steps_per_plane) < _NFILL)
    def _fill():
        p = i // steps_per_plane
        row8 = pos_ref[p]
        for k in range(_BB // 8):
            out_ref[pl.ds(8 * k, 8), :] = row8


def kernel(dummy, position_ids):
    b = dummy.shape[0]
    _, c, one, s = position_ids.shape
    table = lax.bitcast_convert_type(position_ids.reshape(c, 1, s), jnp.bfloat16)
    table8 = jnp.broadcast_to(table, (c, 8, s))
    idx2d = dummy.reshape(1, b)
    grid = (c * b) // _BB
    out = pl.pallas_call(
        _bcast_kernel,
        grid=(grid,),
        in_specs=[
            pl.BlockSpec((1, b), lambda i: (0, 0)),
            pl.BlockSpec((c, 8, s), lambda i: (0, 0, 0)),
        ],
        out_specs=pl.BlockSpec((_BB, s), lambda i: (i, 0)),
        out_shape=jax.ShapeDtypeStruct((c * b, s), jnp.bfloat16),
    )(idx2d, table8)
    out16 = lax.bitcast_convert_type(out, position_ids.dtype)
    out3 = out16.reshape(c, b, s)
    return jnp.transpose(out3, (1, 0, 2)).reshape(b, c, one, s)


# speedup vs baseline: 1.0165x; 1.0165x over previous
import jax
import jax.numpy as jnp
from jax import lax
from jax.experimental import pallas as pl

_BB = 128
_NFILL = 2


def _bcast_kernel(dummy_ref, pos_ref, out_ref):
    del dummy_ref
    i = pl.program_id(0)
    n = pl.num_programs(0)
    steps_per_plane = n // pos_ref.shape[0]

    @pl.when(lax.rem(i, steps_per_plane) < _NFILL)
    def _fill():
        p = i // steps_per_plane
        row8 = pos_ref[p]
        for k in range(_BB // 8):
            out_ref[pl.ds(8 * k, 8), :] = row8


def kernel(dummy, position_ids):
    b = dummy.shape[0]
    _, c, one, s = position_ids.shape
    table = lax.bitcast_convert_type(position_ids.reshape(c, 1, s), jnp.bfloat16)
    table8 = jnp.broadcast_to(table, (c, 8, s))
    idx2d = dummy.reshape(1, b)
    grid = (c * b) // _BB
    out = pl.pallas_call(
        _bcast_kernel,
        grid=(grid,),
        in_specs=[
            pl.BlockSpec((1, b), lambda i: (0, 0)),
            pl.BlockSpec((c, 8, s), lambda i: (0, 0, 0)),
        ],
        out_specs=pl.BlockSpec((_BB, s), lambda i: (i, 0)),
        out_shape=jax.ShapeDtypeStruct((c * b, s), jnp.bfloat16),
    )(idx2d, table8)
    out16 = lax.bitcast_convert_type(out, position_ids.dtype)
    out3 = out16.reshape(c, b, s)
    return jnp.transpose(out3, (1, 0, 2)).reshape(b, c, one, s)
